# baseline (speedup 1.0000x reference)
import functools

import jax
import jax.numpy as jnp
from jax import lax
from jax.experimental import pallas as pl
from jax.experimental.pallas import tpu as pltpu

_BLOCK_ROWS = 16384


def _linear_kernel(x_ref, w_ref, b_ref, o_ref):
    acc = lax.dot_general(
        x_ref[...],
        w_ref[...],
        (((1,), (1,)), ((), ())),
        preferred_element_type=jnp.float32,
    )
    o_ref[...] = (acc + b_ref[...]).astype(o_ref.dtype)


@functools.partial(jax.jit, static_argnames=("block_rows",))
def _native_linear(x, weight, bias2d, *, block_rows):
    rows, in_f = x.shape
    out_f = weight.shape[0]
    grid = (pl.cdiv(rows, block_rows),)
    return pl.pallas_call(
        _linear_kernel,
        out_shape=jax.ShapeDtypeStruct((rows, out_f), x.dtype),
        grid=grid,
        in_specs=[
            pl.BlockSpec((block_rows, in_f), lambda i: (i, 0)),
            pl.BlockSpec((weight.shape[0], in_f), lambda i: (0, 0)),
            pl.BlockSpec((1, out_f), lambda i: (0, 0)),
        ],
        out_specs=pl.BlockSpec((block_rows, out_f), lambda i: (i, 0)),
        compiler_params=pltpu.CompilerParams(
            dimension_semantics=("arbitrary",)),
    )(x, weight, bias2d)


def kernel(x, weight, bias):
    return _native_linear(x, weight, bias.reshape(1, -1),
                          block_rows=_BLOCK_ROWS)


# speedup vs baseline: 1.0030x; 1.0030x over previous
import functools

import jax
import jax.numpy as jnp
from jax import lax
from jax.experimental import pallas as pl
from jax.experimental.pallas import tpu as pltpu

_CHUNK = 16384
_DEPTH = 3


def _make_pipeline_kernel(n_full, tail):
    def _pipeline_kernel(x_hbm, w_ref, b_ref, o_hbm,
                         x_bufs, y_bufs, x_tail, y_tail,
                         in_sems, out_sems, tail_sems):
        def start_load(i, slot):
            pltpu.make_async_copy(
                x_hbm.at[pl.ds(i * _CHUNK, _CHUNK), :],
                x_bufs.at[slot],
                in_sems.at[slot],
            ).start()

        def linear(xb):
            acc = lax.dot_general(
                xb, w_ref[...], (((1,), (1,)), ((), ())),
                preferred_element_type=jnp.float32,
            )
            return (acc + b_ref[...]).astype(xb.dtype)

        if tail:
            pltpu.make_async_copy(
                x_hbm.at[pl.ds(n_full * _CHUNK, tail), :],
                x_tail, tail_sems.at[0],
            ).start()
        for s in range(min(_DEPTH, n_full)):
            start_load(s, s)

        def body(i, carry):
            slot = lax.rem(i, _DEPTH)

            @pl.when(i >= _DEPTH)
            def _wait_prev_store():
                pltpu.make_async_copy(
                    y_bufs.at[slot], y_bufs.at[slot], out_sems.at[slot]
                ).wait()

            pltpu.make_async_copy(
                x_bufs.at[slot], x_bufs.at[slot], in_sems.at[slot]
            ).wait()

            y_bufs[slot, :, :] = linear(x_bufs[slot, :, :])

            pltpu.make_async_copy(
                y_bufs.at[slot],
                o_hbm.at[pl.ds(i * _CHUNK, _CHUNK), :],
                out_sems.at[slot],
            ).start(priority=1)

            @pl.when(i + _DEPTH < n_full)
            def _next_load():
                start_load(i + _DEPTH, slot)

            return carry

        lax.fori_loop(0, n_full, body, 0)

        if tail:
            pltpu.make_async_copy(x_tail, x_tail, tail_sems.at[0]).wait()
            y_tail[...] = linear(x_tail[...])
            pltpu.make_async_copy(
                y_tail,
                o_hbm.at[pl.ds(n_full * _CHUNK, tail), :],
                tail_sems.at[1],
            ).start(priority=1)

        for s in range(min(_DEPTH, n_full)):
            pltpu.make_async_copy(
                y_bufs.at[s], y_bufs.at[s], out_sems.at[s]
            ).wait()
        if tail:
            pltpu.make_async_copy(y_tail, y_tail, tail_sems.at[1]).wait()

    return _pipeline_kernel


@functools.partial(jax.jit, static_argnames=("n_full", "tail"))
def _native_linear(x, weight, bias2d, *, n_full, tail):
    rows, in_f = x.shape
    out_f = weight.shape[0]
    return pl.pallas_call(
        _make_pipeline_kernel(n_full, tail),
        out_shape=jax.ShapeDtypeStruct((rows, out_f), x.dtype),
        in_specs=[
            pl.BlockSpec(memory_space=pl.ANY),
            pl.BlockSpec(memory_space=pltpu.MemorySpace.VMEM),
            pl.BlockSpec(memory_space=pltpu.MemorySpace.VMEM),
        ],
        out_specs=pl.BlockSpec(memory_space=pl.ANY),
        scratch_shapes=[
            pltpu.VMEM((_DEPTH, _CHUNK, in_f), x.dtype),
            pltpu.VMEM((_DEPTH, _CHUNK, out_f), x.dtype),
            pltpu.VMEM((max(tail, 8), in_f), x.dtype),
            pltpu.VMEM((max(tail, 8), out_f), x.dtype),
            pltpu.SemaphoreType.DMA((_DEPTH,)),
            pltpu.SemaphoreType.DMA((_DEPTH,)),
            pltpu.SemaphoreType.DMA((2,)),
        ],
    )(x, weight, bias2d)


def kernel(x, weight, bias):
    B = x.shape[0]
    return _native_linear(x, weight, bias.reshape(1, -1),
                          n_full=B // _CHUNK, tail=B % _CHUNK)


# speedup vs baseline: 63.4226x; 63.2309x over previous
import functools

import jax
import jax.numpy as jnp
from jax import lax
from jax.experimental import pallas as pl
from jax.experimental.pallas import tpu as pltpu

_BLOCK_N = 131072


def _linear_t_kernel(x_ref, w_ref, b_ref, o_ref):
    acc = jnp.dot(w_ref[...], x_ref[...],
                  preferred_element_type=jnp.float32)
    o_ref[...] = (acc + b_ref[...]).astype(o_ref.dtype)


@functools.partial(jax.jit, static_argnames=("block_n",))
def _linear_t(x_t, weight, bias2d, *, block_n):
    in_f, cols = x_t.shape
    out_f = weight.shape[0]
    grid = (pl.cdiv(cols, block_n),)
    return pl.pallas_call(
        _linear_t_kernel,
        out_shape=jax.ShapeDtypeStruct((out_f, cols), x_t.dtype),
        grid=grid,
        in_specs=[
            pl.BlockSpec((in_f, block_n), lambda i: (0, i)),
            pl.BlockSpec((out_f, in_f), lambda i: (0, 0)),
            pl.BlockSpec((out_f, 1), lambda i: (0, 0)),
        ],
        out_specs=pl.BlockSpec((out_f, block_n), lambda i: (0, i)),
        compiler_params=pltpu.CompilerParams(
            dimension_semantics=("arbitrary",)),
    )(x_t, weight, bias2d)


def kernel(x, weight, bias):
    x_t = x.T
    y_t = _linear_t(x_t, weight, bias.reshape(-1, 1), block_n=_BLOCK_N)
    return y_t.T


# speedup vs baseline: 74.6728x; 1.1774x over previous
import functools

import jax
import jax.numpy as jnp
from jax import lax
from jax.experimental import pallas as pl
from jax.experimental.pallas import tpu as pltpu

_BLOCK_N = 262144


def _linear_t_kernel(x_ref, w_ref, b_ref, o_ref):
    acc = jnp.dot(w_ref[...], x_ref[...],
                  preferred_element_type=jnp.float32)
    o_ref[...] = (acc + b_ref[...]).astype(o_ref.dtype)


@functools.partial(jax.jit, static_argnames=("block_n",))
def _linear_t(x_t, weight, bias2d, *, block_n):
    in_f, cols = x_t.shape
    out_f = weight.shape[0]
    grid = (pl.cdiv(cols, block_n),)
    return pl.pallas_call(
        _linear_t_kernel,
        out_shape=jax.ShapeDtypeStruct((out_f, cols), x_t.dtype),
        grid=grid,
        in_specs=[
            pl.BlockSpec((in_f, block_n), lambda i: (0, i)),
            pl.BlockSpec((out_f, in_f), lambda i: (0, 0)),
            pl.BlockSpec((out_f, 1), lambda i: (0, 0)),
        ],
        out_specs=pl.BlockSpec((out_f, block_n), lambda i: (0, i)),
        compiler_params=pltpu.CompilerParams(
            dimension_semantics=("arbitrary",)),
    )(x_t, weight, bias2d)


def kernel(x, weight, bias):
    x_t = x.T
    y_t = _linear_t(x_t, weight, bias.reshape(-1, 1), block_n=_BLOCK_N)
    return y_t.T


# speedup vs baseline: 77.5841x; 1.0390x over previous
import functools

import jax
import jax.numpy as jnp
from jax import lax
from jax.experimental import pallas as pl
from jax.experimental.pallas import tpu as pltpu

_BLOCK_N = 524288


def _linear_t_kernel(x_ref, w_ref, b_ref, o_ref):
    acc = jnp.dot(w_ref[...], x_ref[...],
                  preferred_element_type=jnp.float32)
    o_ref[...] = (acc + b_ref[...]).astype(o_ref.dtype)


@functools.partial(jax.jit, static_argnames=("block_n",))
def _linear_t(x_t, weight, bias2d, *, block_n):
    in_f, cols = x_t.shape
    out_f = weight.shape[0]
    grid = (pl.cdiv(cols, block_n),)
    return pl.pallas_call(
        _linear_t_kernel,
        out_shape=jax.ShapeDtypeStruct((out_f, cols), x_t.dtype),
        grid=grid,
        in_specs=[
            pl.BlockSpec((in_f, block_n), lambda i: (0, i)),
            pl.BlockSpec((out_f, in_f), lambda i: (0, 0)),
            pl.BlockSpec((out_f, 1), lambda i: (0, 0)),
        ],
        out_specs=pl.BlockSpec((out_f, block_n), lambda i: (0, i)),
        compiler_params=pltpu.CompilerParams(
            dimension_semantics=("arbitrary",)),
    )(x_t, weight, bias2d)


def kernel(x, weight, bias):
    x_t = x.T
    y_t = _linear_t(x_t, weight, bias.reshape(-1, 1), block_n=_BLOCK_N)
    return y_t.T
